# baseline (speedup 1.0000x reference)
import functools

import jax
import jax.numpy as jnp
from jax import lax
from jax.experimental import pallas as pl
from jax.experimental.pallas import tpu as pltpu
from jax.experimental.pallas import tpu_sc as plsc

_B, _T, _E = 4, 8192, 2048
_NW = 32
_R = 8
_ROWS_W = _T // _NW
_VECS = _E // 16


def _sc_body(x_hbm, pos_hbm, out_hbm, posv, xv):
    wid = lax.axis_index("s") * 2 + lax.axis_index("c")
    t_base = wid * _ROWS_W

    def chunk(c, carry):
        t0 = t_base + c * _R
        pltpu.sync_copy(pos_hbm.at[pl.ds(t0, _R)], posv)
        for b in range(_B):
            pltpu.sync_copy(x_hbm.at[b, pl.ds(t0, _R)], xv)

            def add(i, acc):
                r = i // _VECS
                col = (i % _VECS) * 16
                xv[r, pl.ds(col, 16)] = xv[r, pl.ds(col, 16)] + posv[r, pl.ds(col, 16)]
                return acc

            lax.fori_loop(0, _R * _VECS, add, 0)
            pltpu.sync_copy(xv, out_hbm.at[b, pl.ds(t0, _R)])
        return carry

    lax.fori_loop(0, _ROWS_W // _R, chunk, 0)


def _sc_kernel(x, pos_table):
    mesh = plsc.VectorSubcoreMesh(core_axis_name="c", subcore_axis_name="s")
    f = pl.kernel(
        _sc_body,
        out_type=jax.ShapeDtypeStruct((_B, _T, _E), jnp.float32),
        mesh=mesh,
        scratch_types=[
            pltpu.VMEM((_R, _E), jnp.float32),
            pltpu.VMEM((_R, _E), jnp.float32),
        ],
    )
    return f(x, pos_table)


def kernel(x, pos_table):
    return _sc_kernel(x, pos_table)


# speedup vs baseline: 1.2791x; 1.2791x over previous
import jax
import jax.numpy as jnp
from jax import lax
from jax.experimental import pallas as pl
from jax.experimental.pallas import tpu as pltpu
from jax.experimental.pallas import tpu_sc as plsc

_B, _T, _E = 4, 8192, 2048
_NW = 32
_R = 8
_VECS = _E // 16


def _sc_body(x_hbm, pos_hbm, out_hbm, posv, xv0, xv1, psem, lsem0, lsem1, ssem0, ssem1):
    wid = lax.axis_index("s") * 2 + lax.axis_index("c")
    t_base = wid * (_T // _NW)
    bufs = (xv0, xv1)
    lsems = (lsem0, lsem1)
    ssems = (ssem0, ssem1)

    def add_rows(buf):
        def add_row(r, acc):
            for j in range(_VECS):
                sl = pl.ds(j * 16, 16)
                buf[r, sl] = buf[r, sl] + posv[r, sl]
            return acc
        lax.fori_loop(0, _R, add_row, 0)

    def chunk(c, carry):
        t0 = t_base + c * _R
        pltpu.make_async_copy(pos_hbm.at[pl.ds(t0, _R)], posv, psem).start()
        pltpu.make_async_copy(x_hbm.at[0, pl.ds(t0, _R)], bufs[0], lsems[0]).start()
        pltpu.make_async_copy(pos_hbm.at[pl.ds(t0, _R)], posv, psem).wait()
        for b in range(_B):
            p = b & 1
            q = p ^ 1
            if b + 1 < _B:
                if b >= 1:
                    pltpu.make_async_copy(bufs[q], out_hbm.at[b - 1, pl.ds(t0, _R)], ssems[q]).wait()
                pltpu.make_async_copy(x_hbm.at[b + 1, pl.ds(t0, _R)], bufs[q], lsems[q]).start()
            pltpu.make_async_copy(x_hbm.at[b, pl.ds(t0, _R)], bufs[p], lsems[p]).wait()
            add_rows(bufs[p])
            pltpu.make_async_copy(bufs[p], out_hbm.at[b, pl.ds(t0, _R)], ssems[p]).start()
        pltpu.make_async_copy(bufs[0], out_hbm.at[_B - 2, pl.ds(t0, _R)], ssems[0]).wait()
        pltpu.make_async_copy(bufs[1], out_hbm.at[_B - 1, pl.ds(t0, _R)], ssems[1]).wait()
        return carry

    lax.fori_loop(0, _T // _NW // _R, chunk, 0)


def _sc_kernel(x, pos_table):
    mesh = plsc.VectorSubcoreMesh(core_axis_name="c", subcore_axis_name="s")
    f = pl.kernel(
        _sc_body,
        out_type=jax.ShapeDtypeStruct((_B, _T, _E), jnp.float32),
        mesh=mesh,
        scratch_types=[
            pltpu.VMEM((_R, _E), jnp.float32),
            pltpu.VMEM((_R, _E), jnp.float32),
            pltpu.VMEM((_R, _E), jnp.float32),
            pltpu.SemaphoreType.DMA,
            pltpu.SemaphoreType.DMA,
            pltpu.SemaphoreType.DMA,
            pltpu.SemaphoreType.DMA,
            pltpu.SemaphoreType.DMA,
        ],
    )
    return f(x, pos_table)


def kernel(x, pos_table):
    return _sc_kernel(x, pos_table)


# speedup vs baseline: 2.4782x; 1.9374x over previous
import jax
import jax.numpy as jnp
from jax import lax
from jax.experimental import pallas as pl
from jax.experimental.pallas import tpu as pltpu
from jax.experimental.pallas import tpu_sc as plsc

_B, _T, _E = 4, 8192, 2048
_T_SC = 1536
_T_TC = _T - _T_SC
_TS = 512
_NW = 32
_R = 8
_VECS = _E // 16


def _tc_body(x_ref, pos_ref, o_ref):
    o_ref[...] = x_ref[...] + pos_ref[...]


def _tc_part(x, pos_table):
    return pl.pallas_call(
        _tc_body,
        grid=(_T_TC // _TS, _B),
        in_specs=[
            pl.BlockSpec((1, _TS, _E), lambda t, b: (b, t, 0)),
            pl.BlockSpec((_TS, _E), lambda t, b: (t, 0)),
        ],
        out_specs=pl.BlockSpec((1, _TS, _E), lambda t, b: (b, t, 0)),
        out_shape=jax.ShapeDtypeStruct((_B, _T_TC, _E), x.dtype),
    )(x, pos_table)


def _sc_body(x_hbm, pos_hbm, out_hbm, posv, xv0, xv1, psem, lsem0, lsem1, ssem0, ssem1):
    wid = lax.axis_index("s") * 2 + lax.axis_index("c")
    rows_w = _T_SC // _NW
    t_base = _T_TC + wid * rows_w
    o_base = wid * rows_w
    bufs = (xv0, xv1)
    lsems = (lsem0, lsem1)
    ssems = (ssem0, ssem1)

    def add_rows(buf):
        def add_row(r, acc):
            for j in range(_VECS):
                sl = pl.ds(j * 16, 16)
                buf[r, sl] = buf[r, sl] + posv[r, sl]
            return acc
        lax.fori_loop(0, _R, add_row, 0)

    def chunk(c, carry):
        t0 = t_base + c * _R
        o0 = o_base + c * _R
        pltpu.make_async_copy(pos_hbm.at[pl.ds(t0, _R)], posv, psem).start()
        pltpu.make_async_copy(x_hbm.at[0, pl.ds(t0, _R)], bufs[0], lsems[0]).start()
        pltpu.make_async_copy(pos_hbm.at[pl.ds(t0, _R)], posv, psem).wait()
        for b in range(_B):
            p = b & 1
            q = p ^ 1
            if b + 1 < _B:
                if b >= 1:
                    pltpu.make_async_copy(bufs[q], out_hbm.at[b - 1, pl.ds(o0, _R)], ssems[q]).wait()
                pltpu.make_async_copy(x_hbm.at[b + 1, pl.ds(t0, _R)], bufs[q], lsems[q]).start()
            pltpu.make_async_copy(x_hbm.at[b, pl.ds(t0, _R)], bufs[p], lsems[p]).wait()
            add_rows(bufs[p])
            pltpu.make_async_copy(bufs[p], out_hbm.at[b, pl.ds(o0, _R)], ssems[p]).start()
        pltpu.make_async_copy(bufs[0], out_hbm.at[_B - 2, pl.ds(o0, _R)], ssems[0]).wait()
        pltpu.make_async_copy(bufs[1], out_hbm.at[_B - 1, pl.ds(o0, _R)], ssems[1]).wait()
        return carry

    lax.fori_loop(0, rows_w // _R, chunk, 0)


def _sc_part(x, pos_table):
    mesh = plsc.VectorSubcoreMesh(core_axis_name="c", subcore_axis_name="s")
    f = pl.kernel(
        _sc_body,
        out_type=jax.ShapeDtypeStruct((_B, _T_SC, _E), jnp.float32),
        mesh=mesh,
        scratch_types=[
            pltpu.VMEM((_R, _E), jnp.float32),
            pltpu.VMEM((_R, _E), jnp.float32),
            pltpu.VMEM((_R, _E), jnp.float32),
            pltpu.SemaphoreType.DMA,
            pltpu.SemaphoreType.DMA,
            pltpu.SemaphoreType.DMA,
            pltpu.SemaphoreType.DMA,
            pltpu.SemaphoreType.DMA,
        ],
    )
    return f(x, pos_table)


def kernel(x, pos_table):
    out_sc = _sc_part(x, pos_table)
    out_tc = _tc_part(x, pos_table)
    return jnp.concatenate([out_tc, out_sc], axis=1)


# speedup vs baseline: 5.0309x; 2.0301x over previous
import jax
import jax.numpy as jnp
from jax.experimental import pallas as pl

_TS = 1024


def _add_kernel(x_ref, pos_ref, o_ref):
    o_ref[...] = x_ref[...] + pos_ref[...]


def kernel(x, pos_table):
    B, T, E = x.shape
    grid = (T // _TS, B)
    return pl.pallas_call(
        _add_kernel,
        grid=grid,
        in_specs=[
            pl.BlockSpec((1, _TS, E), lambda t, b: (b, t, 0)),
            pl.BlockSpec((_TS, E), lambda t, b: (t, 0)),
        ],
        out_specs=pl.BlockSpec((1, _TS, E), lambda t, b: (b, t, 0)),
        out_shape=jax.ShapeDtypeStruct((B, T, E), x.dtype),
    )(x, pos_table)
